# Initial kernel scaffold; baseline (speedup 1.0000x reference)
import jax
import jax.numpy as jnp
from jax.experimental import pallas as pl


def kernel(x, t, pad, token_table, tok_gamma, tok_beta, temporal_table, tmp_gamma, tmp_beta):
    raise NotImplementedError("write your pallas kernel here")


# speedup vs baseline: 2.0444x; 2.0444x over previous
import functools

import jax
import jax.numpy as jnp
from jax import lax
from jax.experimental import pallas as pl
from jax.experimental.pallas import tpu as pltpu
from jax.experimental.pallas import tpu_sc as plsc

CH = 64
EPS = 1e-5
_NC = 2
_NS = 16
_W = _NC * _NS

_C = 512
_G = 128
_NG = _C // _G


def _rsqrt(v):
    i = lax.bitcast_convert_type(v, jnp.int32)
    i = jnp.int32(0x5F3759DF) - lax.shift_right_arithmetic(i, 1)
    y = lax.bitcast_convert_type(i, jnp.float32)
    for _ in range(3):
        y = y * (1.5 - 0.5 * v * y * y)
    return y


def _lane_total(v):
    c1 = plsc.cumsum(v)
    c2 = lax.rev(plsc.cumsum(lax.rev(v, (0,))), (0,))
    return (c1 + c2) - v


def _ln_loop(buf, g4, b4, nrows, tmp=None):
    def row_body(r, carry):
        a = [buf[r, pl.ds(16 * i, 16)] for i in range(4)]
        s = (a[0] + a[1]) + (a[2] + a[3])
        q = (a[0] * a[0] + a[1] * a[1]) + (a[2] * a[2] + a[3] * a[3])
        mean = _lane_total(s) * (1.0 / CH)
        var = _lane_total(q) * (1.0 / CH) - mean * mean
        rs = _rsqrt(var + EPS)
        for i in range(4):
            val = (a[i] - mean) * (rs * g4[i]) + b4[i]
            if tmp is not None:
                val = val + tmp[r, pl.ds(16 * i, 16)]
            buf[r, pl.ds(16 * i, 16)] = val
        return carry

    lax.fori_loop(0, nrows, row_body, 0, unroll=4)


def _wid():
    return lax.axis_index("s") * _NC + lax.axis_index("c")


def _tmp_norm_body(tbl_hbm, g_hbm, b_hbm, out_hbm, buf, g_v, b_v):
    rows = tbl_hbm.shape[0] // _W
    base = _wid() * rows
    pltpu.sync_copy(g_hbm, g_v)
    pltpu.sync_copy(b_hbm, b_v)
    pltpu.sync_copy(tbl_hbm.at[pl.ds(base, rows)], buf)
    g4 = [g_v[pl.ds(16 * i, 16)] for i in range(4)]
    b4 = [b_v[pl.ds(16 * i, 16)] for i in range(4)]
    _ln_loop(buf, g4, b4, rows)
    pltpu.sync_copy(buf, out_hbm.at[pl.ds(base, rows)])


def _main_body(nchunk, tok_hbm, xi_hbm, ntmp_hbm, ti_hbm, g_hbm, b_hbm,
               out_hbm, xi_v, ti_v, tok_v, tmp_v, g_v, b_v, sem_a, sem_b):
    wid = _wid()
    pltpu.sync_copy(g_hbm, g_v)
    pltpu.sync_copy(b_hbm, b_v)
    g4 = [g_v[pl.ds(16 * i, 16)] for i in range(4)]
    b4 = [b_v[pl.ds(16 * i, 16)] for i in range(4)]
    nr_blocks = nchunk * _NG

    def chunk(c, carry):
        rb = wid * nr_blocks + c * _NG
        pltpu.sync_copy(xi_hbm.at[pl.ds(rb, _NG)], xi_v)
        pltpu.sync_copy(ti_hbm.at[pl.ds(rb, _NG)], ti_v)
        cps = []
        for j in range(_NG):
            cps.append(pltpu.async_copy(
                tok_hbm.at[xi_v.at[j]], tok_v.at[pl.ds(j * _G, _G)], sem_a))
            cps.append(pltpu.async_copy(
                ntmp_hbm.at[ti_v.at[j]], tmp_v.at[pl.ds(j * _G, _G)], sem_b))
        for cp in cps:
            cp.wait()
        _ln_loop(tok_v, g4, b4, _C, tmp=tmp_v)
        base = wid * (nchunk * _C) + c * _C
        pltpu.sync_copy(tok_v, out_hbm.at[pl.ds(base, _C)])
        return carry

    lax.fori_loop(0, nchunk, chunk, 0)


def kernel(x, t, pad, token_table, tok_gamma, tok_beta, temporal_table,
           tmp_gamma, tmp_beta):
    del pad
    n = x.size
    assert n % (_W * _C) == 0
    nchunk = n // (_W * _C)
    xf = x.reshape(n // _G, _G).astype(jnp.int32)
    tf = t.reshape(n // _G, _G).astype(jnp.int32)
    mesh = plsc.VectorSubcoreMesh(core_axis_name="c", subcore_axis_name="s")
    params = pltpu.CompilerParams(
        needs_layout_passes=False, use_tc_tiling_on_sc=False)

    tmp_norm = pl.kernel(
        _tmp_norm_body,
        out_type=jax.ShapeDtypeStruct(temporal_table.shape, jnp.float32),
        mesh=mesh,
        compiler_params=params,
        scratch_types=[
            pltpu.VMEM((temporal_table.shape[0] // _W, CH), jnp.float32),
            pltpu.VMEM((CH,), jnp.float32),
            pltpu.VMEM((CH,), jnp.float32),
        ],
    )
    ntmp = tmp_norm(temporal_table, tmp_gamma, tmp_beta)

    main = pl.kernel(
        functools.partial(_main_body, nchunk),
        out_type=jax.ShapeDtypeStruct((n, CH), jnp.float32),
        mesh=mesh,
        compiler_params=params,
        scratch_types=[
            pltpu.VMEM((_NG, _G), jnp.int32),
            pltpu.VMEM((_NG, _G), jnp.int32),
            pltpu.VMEM((_C, CH), jnp.float32),
            pltpu.VMEM((_C, CH), jnp.float32),
            pltpu.VMEM((CH,), jnp.float32),
            pltpu.VMEM((CH,), jnp.float32),
            pltpu.SemaphoreType.DMA,
            pltpu.SemaphoreType.DMA,
        ],
    )
    out = main(token_table, xf, ntmp, tf, tok_gamma, tok_beta)
    return out.reshape(x.shape + (CH,))


# speedup vs baseline: 2.6146x; 1.2789x over previous
import functools

import jax
import jax.numpy as jnp
from jax import lax
from jax.experimental import pallas as pl
from jax.experimental.pallas import tpu as pltpu
from jax.experimental.pallas import tpu_sc as plsc

CH = 64
EPS = 1e-5
_NC = 2
_NS = 16
_W = _NC * _NS

_C = 256
_G = 128
_NG = _C // _G


def _rsqrt(v):
    i = lax.bitcast_convert_type(v, jnp.int32)
    i = jnp.int32(0x5F3759DF) - lax.shift_right_arithmetic(i, 1)
    y = lax.bitcast_convert_type(i, jnp.float32)
    for _ in range(2):
        y = y * (1.5 - 0.5 * v * y * y)
    return y


def _ln_loop(buf, g4, b4, nrows, m_sc, r_sc, tmp=None):
    @plsc.parallel_loop(0, nrows, unroll=8)
    def _(r):
        a = [buf[r, pl.ds(16 * i, 16)] for i in range(4)]
        s = (a[0] + a[1]) + (a[2] + a[3])
        q = (a[0] * a[0] + a[1] * a[1]) + (a[2] * a[2] + a[3] * a[3])
        mean = jnp.sum(s) * (1.0 / CH)
        var = jnp.sum(q) * (1.0 / CH) - mean * mean
        m_sc[r] = mean
        r_sc[r] = _rsqrt(var + EPS)

    @plsc.parallel_loop(0, nrows, unroll=4)
    def _(r):
        m = m_sc[r]
        rs = r_sc[r]
        for i in range(4):
            val = (buf[r, pl.ds(16 * i, 16)] - m) * (rs * g4[i]) + b4[i]
            if tmp is not None:
                val = val + tmp[r, pl.ds(16 * i, 16)]
            buf[r, pl.ds(16 * i, 16)] = val


def _wid():
    return lax.axis_index("s") * _NC + lax.axis_index("c")


def _tmp_norm_body(tbl_hbm, g_hbm, b_hbm, out_hbm, buf, g_v, b_v, m_sc, r_sc):
    rows = tbl_hbm.shape[0] // _W
    base = _wid() * rows
    pltpu.sync_copy(g_hbm, g_v)
    pltpu.sync_copy(b_hbm, b_v)
    pltpu.sync_copy(tbl_hbm.at[pl.ds(base, rows)], buf)
    g4 = [g_v[pl.ds(16 * i, 16)] for i in range(4)]
    b4 = [b_v[pl.ds(16 * i, 16)] for i in range(4)]
    _ln_loop(buf, g4, b4, rows, m_sc, r_sc)
    pltpu.sync_copy(buf, out_hbm.at[pl.ds(base, rows)])


def _main_body(nchunk, tok_hbm, idx_hbm, ntmp_hbm, g_hbm, b_hbm, out_hbm,
               ib0, ib1, tok0, tok1, tmp0, tmp1, g_v, b_v, m_sc, r_sc,
               si0, si1, sg0, sg1, so0, so1):
    wid = _wid()
    pltpu.sync_copy(g_hbm, g_v)
    pltpu.sync_copy(b_hbm, b_v)
    g4 = [g_v[pl.ds(16 * i, 16)] for i in range(4)]
    b4 = [b_v[pl.ds(16 * i, 16)] for i in range(4)]

    ib = [ib0, ib1]
    tok = [tok0, tok1]
    tmp = [tmp0, tmp1]
    si = [si0, si1]
    sg = [sg0, sg1]
    so = [so0, so1]

    ibase = wid * (nchunk * _NG)
    obase = wid * (nchunk * _C)

    def issue_gathers(h, b):
        for j in range(_NG):
            pltpu.async_copy(tok_hbm.at[ib[b].at[j, 0]],
                             tok[b].at[pl.ds(j * _G, _G)], sg[b])
            pltpu.async_copy(ntmp_hbm.at[ib[b].at[j, 1]],
                             tmp[b].at[pl.ds(j * _G, _G)], sg[b])

    def wait_gathers(b):
        pltpu.make_async_copy(out_hbm.at[pl.ds(0, _C)], tok[b], sg[b]).wait()
        pltpu.make_async_copy(out_hbm.at[pl.ds(0, _C)], tmp[b], sg[b]).wait()

    def issue_idx(h, b):
        return pltpu.async_copy(idx_hbm.at[pl.ds(ibase + h * _NG, _NG)],
                                ib[b], si[b])

    def wait_idx(b):
        pltpu.make_async_copy(idx_hbm.at[pl.ds(0, _NG)], ib[b], si[b]).wait()

    def wait_out(b):
        pltpu.make_async_copy(out_hbm.at[pl.ds(0, _C)], tok[b], so[b]).wait()

    pltpu.sync_copy(idx_hbm.at[pl.ds(ibase, _NG)], ib[0])
    issue_gathers(0, 0)
    issue_idx(1, 1)

    def half(g, b):
        nb = 1 - b
        wait_gathers(b)
        issue_idx(jnp.minimum(g + 2, nchunk - 1), b)
        wait_idx(nb)

        @pl.when(g >= 1)
        def _():
            wait_out(nb)
        issue_gathers(jnp.minimum(g + 1, nchunk - 1), nb)
        _ln_loop(tok[b], g4, b4, _C, m_sc, r_sc, tmp=tmp[b])
        pltpu.async_copy(tok[b], out_hbm.at[pl.ds(obase + g * _C, _C)], so[b])

    def pair(p, carry):
        half(2 * p, 0)
        half(2 * p + 1, 1)
        return carry

    lax.fori_loop(0, nchunk // 2, pair, 0)

    wait_idx(1)
    wait_gathers(0)
    wait_out(1)


def kernel(x, t, pad, token_table, tok_gamma, tok_beta, temporal_table,
           tmp_gamma, tmp_beta):
    del pad
    n = x.size
    assert n % (_W * _C * 2) == 0
    nchunk = n // (_W * _C)
    xf = x.reshape(n // _G, _G).astype(jnp.int32)
    tf = t.reshape(n // _G, _G).astype(jnp.int32)
    idx_both = jnp.stack([xf, tf], axis=1)
    mesh = plsc.VectorSubcoreMesh(core_axis_name="c", subcore_axis_name="s")
    params = pltpu.CompilerParams(
        needs_layout_passes=False, use_tc_tiling_on_sc=False)

    tmp_norm = pl.kernel(
        _tmp_norm_body,
        out_type=jax.ShapeDtypeStruct(temporal_table.shape, jnp.float32),
        mesh=mesh,
        compiler_params=params,
        scratch_types=[
            pltpu.VMEM((temporal_table.shape[0] // _W, CH), jnp.float32),
            pltpu.VMEM((CH,), jnp.float32),
            pltpu.VMEM((CH,), jnp.float32),
            pltpu.SMEM((temporal_table.shape[0] // _W,), jnp.float32),
            pltpu.SMEM((temporal_table.shape[0] // _W,), jnp.float32),
        ],
    )
    ntmp = tmp_norm(temporal_table, tmp_gamma, tmp_beta)

    main = pl.kernel(
        functools.partial(_main_body, nchunk),
        out_type=jax.ShapeDtypeStruct((n, CH), jnp.float32),
        mesh=mesh,
        compiler_params=params,
        scratch_types=(
            [pltpu.VMEM((_NG, 2, _G), jnp.int32)] * 2
            + [pltpu.VMEM((_C, CH), jnp.float32)] * 4
            + [pltpu.VMEM((CH,), jnp.float32)] * 2
            + [pltpu.SMEM((_C,), jnp.float32)] * 2
            + [pltpu.SemaphoreType.DMA] * 6
        ),
    )
    out = main(token_table, idx_both, ntmp, tok_gamma, tok_beta)
    return out.reshape(x.shape + (CH,))
